# baseline (speedup 1.0000x reference)
import functools

import jax
import jax.numpy as jnp
from jax import lax
from jax.experimental import pallas as pl
from jax.experimental.pallas import tpu as pltpu
from jax.experimental.pallas import tpu_sc as plsc

BETA_C = 0.2
GAMMA_C = 0.95
N_N = 10000
N_E = 160000
D_F = 256

DC = 64
EC = 80
NB = 80
NBLK = N_N // NB
EPT = N_E // 16
ECH = EPT // EC
N_PASS = (D_F // DC) // 2

_mesh = plsc.VectorSubcoreMesh(core_axis_name="c", subcore_axis_name="s")


@functools.partial(
    pl.kernel,
    out_type=jax.ShapeDtypeStruct((N_N, D_F), jnp.float32),
    mesh=_mesh,
    compiler_params=pltpu.CompilerParams(
        use_tc_tiling_on_sc=False, needs_layout_passes=False),
    scratch_types=dict(
        accq=pltpu.VMEM_SHARED((N_N, DC), jnp.float32),
        accac=pltpu.VMEM_SHARED((N_N, DC), jnp.float32),
        zsh=pltpu.VMEM_SHARED((N_N,), jnp.float32),
        zloc=pltpu.VMEM((N_N,), jnp.float32),
        zrows=pltpu.VMEM((16, D_F), jnp.float32),
        zmg=pltpu.VMEM((16,), jnp.float32),
        eb0=pltpu.VMEM((EC, DC), jnp.float32),
        eb1=pltpu.VMEM((EC, DC), jnp.float32),
        srcb=pltpu.VMEM((EC,), jnp.int32),
        dstb=pltpu.VMEM((EC,), jnp.int32),
        qa0=pltpu.VMEM((NB, DC), jnp.float32),
        qa1=pltpu.VMEM((NB, DC), jnp.float32),
        zb=pltpu.VMEM((NB, DC), jnp.float32),
        ob=pltpu.VMEM((NB, DC), jnp.float32),
    ),
)
def _gnn_sc(z_hbm, e_hbm, src_hbm, dst_hbm, out_hbm, *, accq, accac, zsh, zloc,
            zrows, zmg, eb0, eb1, srcb, dstb, qa0, qa1, zb, ob):
    cid = lax.axis_index("c")
    sid = lax.axis_index("s")
    zero16 = jnp.zeros((16,), jnp.float32)

    n_my_blk = (NBLK - sid + 15) // 16

    lane = lax.iota(jnp.int32, 16)

    n_z16 = ((N_N // 16) - sid + 15) // 16

    def zmax_blk(i, _):
        r0 = (sid + 16 * i) * 16
        pltpu.sync_copy(z_hbm.at[pl.ds(r0, 16)], zrows)

        def dred(d, m):
            v = plsc.load_gather(
                zrows, [lane, jnp.full((16,), d, jnp.int32)])
            return jnp.maximum(m, v)

        m = lax.fori_loop(
            0, D_F, dred, jnp.full((16,), -jnp.inf, jnp.float32),
            unroll=4)
        zmg[pl.ds(0, 16)] = GAMMA_C * m
        pltpu.sync_copy(zmg, zsh.at[pl.ds(r0, 16)])
        return 0

    lax.fori_loop(0, n_z16, zmax_blk, 0)

    plsc.subcore_barrier()
    pltpu.sync_copy(zsh, zloc)

    for p in range(N_PASS):
        k = 2 * p + cid
        col0 = DC * k

        plsc.subcore_barrier()

        def ob_zero(r, _):
            for j in range(DC // 16):
                ob[r, pl.ds(16 * j, 16)] = zero16
            return 0

        lax.fori_loop(0, NB, ob_zero, 0)

        def zero_blk(i, _):
            r0 = (sid + 16 * i) * NB
            pltpu.sync_copy(ob, accq.at[pl.ds(r0, NB)])
            pltpu.sync_copy(ob, accac.at[pl.ds(r0, NB)])
            return 0

        lax.fori_loop(0, n_my_blk, zero_blk, 0)
        plsc.subcore_barrier()

        def edge_chunk(ch, _):
            e0 = sid * EPT + ch * EC
            pltpu.sync_copy(src_hbm.at[pl.ds(e0, EC)], srcb)
            pltpu.sync_copy(dst_hbm.at[pl.ds(e0, EC)], dstb)
            pltpu.sync_copy(e_hbm.at[pl.ds(e0, EC), pl.ds(col0, DC)], eb0)
            pltpu.sync_copy(
                e_hbm.at[pl.ds(e0, EC), pl.ds(D_F + col0, DC)], eb1)

            for g in range(EC // 16):
                rows = lane + g * 16
                src16 = srcb[pl.ds(g * 16, 16)]
                zt = plsc.load_gather(zloc, [src16])

                def col_fma(d, _, rows=rows, zt=zt):
                    dsp = jnp.full((16,), d, jnp.int32)
                    e1v = plsc.load_gather(eb1, [rows, dsp])
                    e0v = plsc.load_gather(eb0, [rows, dsp])
                    plsc.store_scatter(
                        eb0, [rows, dsp], zt * e1v + e0v)
                    return 0

                lax.fori_loop(0, DC, col_fma, 0, unroll=4)
            pltpu.sync_copy(eb0, accq.at[dstb], add=True)
            pltpu.sync_copy(eb1, accac.at[dstb], add=True)
            return 0

        lax.fori_loop(0, ECH, edge_chunk, 0)
        plsc.subcore_barrier()

        def drain_blk(i, _):
            r0 = (sid + 16 * i) * NB
            pltpu.sync_copy(accq.at[pl.ds(r0, NB)], qa0)
            pltpu.sync_copy(accac.at[pl.ds(r0, NB)], qa1)
            pltpu.sync_copy(z_hbm.at[pl.ds(r0, NB), pl.ds(col0, DC)], zb)

            def comb_row(r, _):
                for j in range(DC // 16):
                    sl = pl.ds(16 * j, 16)
                    ob[r, sl] = (BETA_C * zb[r, sl]
                                 + (1.0 - BETA_C) * qa0[r, sl]
                                 / (qa1[r, sl] + 1e-6))
                return 0

            lax.fori_loop(0, NB, comb_row, 0)
            pltpu.sync_copy(ob, out_hbm.at[pl.ds(r0, NB), pl.ds(col0, DC)])
            return 0

        lax.fori_loop(0, n_my_blk, drain_blk, 0)


def kernel(z, e, edge_index):
    e2 = e.reshape(N_E, 2 * D_F)
    ei = edge_index.astype(jnp.int32)
    return _gnn_sc(z, e2, ei[0], ei[1])


# speedup vs baseline: 1.2574x; 1.2574x over previous
import functools

import jax
import jax.numpy as jnp
from jax import lax
from jax.experimental import pallas as pl
from jax.experimental.pallas import tpu as pltpu
from jax.experimental.pallas import tpu_sc as plsc

BETA_C = 0.2
GAMMA_C = 0.95
N_N = 10000
N_E = 160000
D_F = 256

DC = 64
EC = 80
NB = 80
NBLK = N_N // NB
EPT = N_E // 16
ECH = EPT // EC
N_PASS = (D_F // DC) // 2
NBUF = 3

_mesh = plsc.VectorSubcoreMesh(core_axis_name="c", subcore_axis_name="s")

_scratch = dict(
    accq=pltpu.VMEM_SHARED((N_N, DC), jnp.float32),
    accac=pltpu.VMEM_SHARED((N_N, DC), jnp.float32),
    zsh=pltpu.VMEM_SHARED((N_N,), jnp.float32),
    zloc=pltpu.VMEM((N_N,), jnp.float32),
    zrows=pltpu.VMEM((16, D_F), jnp.float32),
    zmg=pltpu.VMEM((16,), jnp.float32),
)
for _b in range(NBUF):
    _scratch[f"eb0_{_b}"] = pltpu.VMEM((EC, DC), jnp.float32)
    _scratch[f"eb1_{_b}"] = pltpu.VMEM((EC, DC), jnp.float32)
    _scratch[f"src_{_b}"] = pltpu.VMEM((EC,), jnp.int32)
    _scratch[f"dst_{_b}"] = pltpu.VMEM((EC,), jnp.int32)
    _scratch[f"isem_{_b}"] = pltpu.SemaphoreType.DMA
    _scratch[f"ssem_{_b}"] = pltpu.SemaphoreType.DMA


@functools.partial(
    pl.kernel,
    out_type=jax.ShapeDtypeStruct((N_N, D_F), jnp.float32),
    mesh=_mesh,
    compiler_params=pltpu.CompilerParams(
        use_tc_tiling_on_sc=False, needs_layout_passes=False),
    scratch_types=_scratch,
)
def _gnn_sc(z_hbm, e_hbm, src_hbm, dst_hbm, out_hbm, *, accq, accac, zsh,
            zloc, zrows, zmg, **bufs):
    cid = lax.axis_index("c")
    sid = lax.axis_index("s")
    lane = lax.iota(jnp.int32, 16)
    zero16 = jnp.zeros((16,), jnp.float32)
    ebs0 = [bufs[f"eb0_{b}"] for b in range(NBUF)]
    ebs1 = [bufs[f"eb1_{b}"] for b in range(NBUF)]
    srcs = [bufs[f"src_{b}"] for b in range(NBUF)]
    dsts = [bufs[f"dst_{b}"] for b in range(NBUF)]
    isems = [bufs[f"isem_{b}"] for b in range(NBUF)]
    ssems = [bufs[f"ssem_{b}"] for b in range(NBUF)]

    n_z16 = ((N_N // 16) - sid + 15) // 16

    def zmax_blk(i, _):
        r0 = (sid + 16 * i) * 16
        pltpu.sync_copy(z_hbm.at[pl.ds(r0, 16)], zrows)

        def dred(d, m):
            v = plsc.load_gather(
                zrows, [lane, jnp.full((16,), d, jnp.int32)])
            return jnp.maximum(m, v)

        m = lax.fori_loop(
            0, D_F, dred, jnp.full((16,), -jnp.inf, jnp.float32),
            unroll=4)
        zmg[pl.ds(0, 16)] = GAMMA_C * m
        pltpu.sync_copy(zmg, zsh.at[pl.ds(r0, 16)])
        return 0

    lax.fori_loop(0, n_z16, zmax_blk, 0)

    plsc.subcore_barrier()
    pltpu.sync_copy(zsh, zloc)

    n_my_blk = (NBLK - sid + 15) // 16

    for p in range(N_PASS):
        k = 2 * p + cid
        col0 = DC * k

        def in_copies(ch, b):
            e0 = sid * EPT + ch * EC
            return (
                pltpu.make_async_copy(
                    src_hbm.at[pl.ds(e0, EC)], srcs[b], isems[b]),
                pltpu.make_async_copy(
                    dst_hbm.at[pl.ds(e0, EC)], dsts[b], isems[b]),
                pltpu.make_async_copy(
                    e_hbm.at[pl.ds(e0, EC), pl.ds(col0, DC)],
                    ebs0[b], isems[b]),
                pltpu.make_async_copy(
                    e_hbm.at[pl.ds(e0, EC), pl.ds(D_F + col0, DC)],
                    ebs1[b], isems[b]),
            )

        def sc_start(b):
            pltpu.async_copy(ebs0[b], accq.at[dsts[b]], ssems[b], add=True)
            pltpu.async_copy(ebs1[b], accac.at[dsts[b]], ssems[b], add=True)

        def sc_wait(b):
            pltpu.make_async_copy(ebs0[b], accq.at[dsts[b]], ssems[b]).wait()
            pltpu.make_async_copy(ebs1[b], accac.at[dsts[b]], ssems[b]).wait()

        plsc.subcore_barrier()

        def zfill(r, _):
            for j in range(DC // 16):
                ebs1[-1][r, pl.ds(16 * j, 16)] = zero16
            return 0

        lax.fori_loop(0, EC, zfill, 0)

        def zero_blk(i, _):
            r0 = (sid + 16 * i) * NB
            pltpu.sync_copy(ebs1[-1], accq.at[pl.ds(r0, NB)])
            pltpu.sync_copy(ebs1[-1], accac.at[pl.ds(r0, NB)])
            return 0

        lax.fori_loop(0, n_my_blk, zero_blk, 0)
        plsc.subcore_barrier()

        def compute(b):
            for g in range(EC // 16):
                rows = lane + g * 16
                src16 = srcs[b][pl.ds(g * 16, 16)]
                zt = plsc.load_gather(zloc, [src16])

                def col_fma(d, _, rows=rows, zt=zt, b=b):
                    dsp = jnp.full((16,), d, jnp.int32)
                    e1v = plsc.load_gather(ebs1[b], [rows, dsp])
                    e0v = plsc.load_gather(ebs0[b], [rows, dsp])
                    plsc.store_scatter(ebs0[b], [rows, dsp],
                                       zt * e1v + e0v)
                    return 0

                lax.fori_loop(0, DC, col_fma, 0, unroll=4)

        for c in in_copies(0, 0):
            c.start()

        def ring(g3, _):
            for b in range(NBUF):
                ch = g3 * NBUF + b

                @pl.when(ch < ECH)
                def _(b=b, ch=ch):
                    nb = (b + 1) % NBUF

                    @pl.when(ch >= 2)
                    def _():
                        sc_wait(nb)

                    @pl.when(ch + 1 < ECH)
                    def _():
                        for c in in_copies(ch + 1, nb):
                            c.start()

                    for c in in_copies(ch, b):
                        c.wait()
                    compute(b)
                    sc_start(b)

            return 0

        lax.fori_loop(0, (ECH + NBUF - 1) // NBUF, ring, 0)
        sc_wait((ECH - 2) % NBUF)
        sc_wait((ECH - 1) % NBUF)

        plsc.subcore_barrier()

        qa0, qa1, zb, obuf = ebs0[0], ebs0[1], ebs0[2], ebs1[0]

        def drain_blk(i, _):
            r0 = (sid + 16 * i) * NB
            pltpu.sync_copy(accq.at[pl.ds(r0, NB)], qa0)
            pltpu.sync_copy(accac.at[pl.ds(r0, NB)], qa1)
            pltpu.sync_copy(z_hbm.at[pl.ds(r0, NB), pl.ds(col0, DC)], zb)

            def comb_row(r, _):
                for j in range(DC // 16):
                    sl = pl.ds(16 * j, 16)
                    obuf[r, sl] = (BETA_C * zb[r, sl]
                                   + (1.0 - BETA_C) * qa0[r, sl]
                                   / (qa1[r, sl] + 1e-6))
                return 0

            lax.fori_loop(0, NB, comb_row, 0)
            pltpu.sync_copy(obuf, out_hbm.at[pl.ds(r0, NB), pl.ds(col0, DC)])
            return 0

        lax.fori_loop(0, n_my_blk, drain_blk, 0)


def kernel(z, e, edge_index):
    e2 = e.reshape(N_E, 2 * D_F)
    ei = edge_index.astype(jnp.int32)
    return _gnn_sc(z, e2, ei[0], ei[1])


# speedup vs baseline: 3.8657x; 3.0745x over previous
import functools

import jax
import jax.numpy as jnp
from jax import lax
from jax.experimental import pallas as pl
from jax.experimental.pallas import tpu as pltpu
from jax.experimental.pallas import tpu_sc as plsc

BETA_C = 0.2
GAMMA_C = 0.95
N_N = 10000
N_E = 160000
D_F = 256

DC = 64
EC = 80
NB = 80
NBLK = N_N // NB
EPT = N_E // 16
ECH = EPT // EC
N_PASS = (D_F // DC) // 2
NBUF = 3

_mesh = plsc.VectorSubcoreMesh(core_axis_name="c", subcore_axis_name="s")

_scratch = dict(
    accq=pltpu.VMEM_SHARED((N_N, DC), jnp.float32),
    accac=pltpu.VMEM_SHARED((N_N, DC), jnp.float32),
    zsh=pltpu.VMEM_SHARED((N_N,), jnp.float32),
    zloc=pltpu.VMEM((N_N,), jnp.float32),
    zrows=pltpu.VMEM((16, D_F), jnp.float32),
    zmg=pltpu.VMEM((16,), jnp.float32),
    ztb=pltpu.VMEM((EC,), jnp.float32),
)
for _b in range(NBUF):
    _scratch[f"eb0_{_b}"] = pltpu.VMEM((EC, DC), jnp.float32)
    _scratch[f"eb1_{_b}"] = pltpu.VMEM((EC, DC), jnp.float32)
    _scratch[f"src_{_b}"] = pltpu.VMEM((EC,), jnp.int32)
    _scratch[f"dst_{_b}"] = pltpu.VMEM((EC,), jnp.int32)
    _scratch[f"isem_{_b}"] = pltpu.SemaphoreType.DMA
    _scratch[f"ssem_{_b}"] = pltpu.SemaphoreType.DMA


@functools.partial(
    pl.kernel,
    out_type=jax.ShapeDtypeStruct((N_N, D_F), jnp.float32),
    mesh=_mesh,
    compiler_params=pltpu.CompilerParams(
        use_tc_tiling_on_sc=False, needs_layout_passes=False),
    scratch_types=_scratch,
)
def _gnn_sc(z_hbm, e_hbm, src_hbm, dst_hbm, out_hbm, *, accq, accac, zsh,
            zloc, zrows, zmg, ztb, **bufs):
    cid = lax.axis_index("c")
    sid = lax.axis_index("s")
    lane = lax.iota(jnp.int32, 16)
    zero16 = jnp.zeros((16,), jnp.float32)
    ebs0 = [bufs[f"eb0_{b}"] for b in range(NBUF)]
    ebs1 = [bufs[f"eb1_{b}"] for b in range(NBUF)]
    srcs = [bufs[f"src_{b}"] for b in range(NBUF)]
    dsts = [bufs[f"dst_{b}"] for b in range(NBUF)]
    isems = [bufs[f"isem_{b}"] for b in range(NBUF)]
    ssems = [bufs[f"ssem_{b}"] for b in range(NBUF)]

    n_z16 = ((N_N // 16) - sid + 15) // 16

    def zmax_blk(i, _):
        r0 = (sid + 16 * i) * 16
        pltpu.sync_copy(z_hbm.at[pl.ds(r0, 16)], zrows)

        def rowred(r, m):
            v = zrows[r, pl.ds(0, 16)]
            for j in range(1, D_F // 16):
                v = jnp.maximum(v, zrows[r, pl.ds(16 * j, 16)])
            return jnp.where(lane == r, jnp.max(v), m)

        m = lax.fori_loop(
            0, 16, rowred, jnp.full((16,), -jnp.inf, jnp.float32))
        zmg[pl.ds(0, 16)] = GAMMA_C * m
        pltpu.sync_copy(zmg, zsh.at[pl.ds(r0, 16)])
        return 0

    lax.fori_loop(0, n_z16, zmax_blk, 0)

    plsc.subcore_barrier()
    pltpu.sync_copy(zsh, zloc)

    n_my_blk = (NBLK - sid + 15) // 16

    for p in range(N_PASS):
        k = 2 * p + cid
        col0 = DC * k

        def in_copies(ch, b):
            e0 = sid * EPT + ch * EC
            return (
                pltpu.make_async_copy(
                    src_hbm.at[pl.ds(e0, EC)], srcs[b], isems[b]),
                pltpu.make_async_copy(
                    dst_hbm.at[pl.ds(e0, EC)], dsts[b], isems[b]),
                pltpu.make_async_copy(
                    e_hbm.at[pl.ds(e0, EC), pl.ds(col0, DC)],
                    ebs0[b], isems[b]),
                pltpu.make_async_copy(
                    e_hbm.at[pl.ds(e0, EC), pl.ds(D_F + col0, DC)],
                    ebs1[b], isems[b]),
            )

        def sc_start(b):
            pltpu.async_copy(ebs0[b], accq.at[dsts[b]], ssems[b], add=True)
            pltpu.async_copy(ebs1[b], accac.at[dsts[b]], ssems[b], add=True)

        def sc_wait(b):
            pltpu.make_async_copy(ebs0[b], accq.at[dsts[b]], ssems[b]).wait()
            pltpu.make_async_copy(ebs1[b], accac.at[dsts[b]], ssems[b]).wait()

        plsc.subcore_barrier()

        def zfill(r, _):
            for j in range(DC // 16):
                ebs1[-1][r, pl.ds(16 * j, 16)] = zero16
            return 0

        lax.fori_loop(0, EC, zfill, 0)

        def zero_blk(i, _):
            r0 = (sid + 16 * i) * NB
            pltpu.sync_copy(ebs1[-1], accq.at[pl.ds(r0, NB)])
            pltpu.sync_copy(ebs1[-1], accac.at[pl.ds(r0, NB)])
            return 0

        lax.fori_loop(0, n_my_blk, zero_blk, 0)
        plsc.subcore_barrier()

        def compute(b):
            for g in range(EC // 16):
                src16 = srcs[b][pl.ds(g * 16, 16)]
                ztb[pl.ds(g * 16, 16)] = plsc.load_gather(zloc, [src16])

            def erow(r, _, b=b):
                ziv = plsc.load_gather(
                    ztb, [jnp.full((16,), r, jnp.int32)])
                for j in range(DC // 16):
                    sl = pl.ds(16 * j, 16)
                    ebs0[b][r, sl] = ziv * ebs1[b][r, sl] + ebs0[b][r, sl]
                return 0

            lax.fori_loop(0, EC, erow, 0, unroll=2)

        for c in in_copies(0, 0):
            c.start()

        def ring(g3, _):
            for b in range(NBUF):
                ch = g3 * NBUF + b

                @pl.when(ch < ECH)
                def _(b=b, ch=ch):
                    nb = (b + 1) % NBUF

                    @pl.when(ch >= 2)
                    def _():
                        sc_wait(nb)

                    @pl.when(ch + 1 < ECH)
                    def _():
                        for c in in_copies(ch + 1, nb):
                            c.start()

                    for c in in_copies(ch, b):
                        c.wait()
                    compute(b)
                    sc_start(b)

            return 0

        lax.fori_loop(0, (ECH + NBUF - 1) // NBUF, ring, 0)
        sc_wait((ECH - 2) % NBUF)
        sc_wait((ECH - 1) % NBUF)

        plsc.subcore_barrier()

        qa0, qa1, zb, obuf = ebs0[0], ebs0[1], ebs0[2], ebs1[0]

        def drain_blk(i, _):
            r0 = (sid + 16 * i) * NB
            pltpu.sync_copy(accq.at[pl.ds(r0, NB)], qa0)
            pltpu.sync_copy(accac.at[pl.ds(r0, NB)], qa1)
            pltpu.sync_copy(z_hbm.at[pl.ds(r0, NB), pl.ds(col0, DC)], zb)

            def comb_row(r, _):
                for j in range(DC // 16):
                    sl = pl.ds(16 * j, 16)
                    obuf[r, sl] = (BETA_C * zb[r, sl]
                                   + (1.0 - BETA_C) * qa0[r, sl]
                                   / (qa1[r, sl] + 1e-6))
                return 0

            lax.fori_loop(0, NB, comb_row, 0)
            pltpu.sync_copy(obuf, out_hbm.at[pl.ds(r0, NB), pl.ds(col0, DC)])
            return 0

        lax.fori_loop(0, n_my_blk, drain_blk, 0)


def kernel(z, e, edge_index):
    e2 = e.reshape(N_E, 2 * D_F)
    ei = edge_index.astype(jnp.int32)
    return _gnn_sc(z, e2, ei[0], ei[1])
